# baseline (speedup 1.0000x reference)
import jax
import jax.numpy as jnp
from jax import lax
from jax.experimental import pallas as pl
from jax.experimental.pallas import tpu as pltpu
from jax.experimental.pallas import tpu_sc as plsc

_NC, _NS, _L = 2, 16, 16
_NW = _NC * _NS
_B = 16384
_BPW = _B // _NW
_D = 32
_CH = 128
_NCH = _BPW // _CH
_GROUPS = _BPW // _L
_BIAS = 3.5


def _sc_body(user_h, item_h, label_h, utab_h, itab_h,
             pred_h, rmse_h, part_h,
             idxu, idxi, ubuf, vbuf, labv, predv, rmsev, sqv, sem):
    wid = lax.axis_index("s") * _NC + lax.axis_index("c")
    base = wid * _BPW

    for j in range(_NCH):
        pltpu.sync_copy(user_h.at[pl.ds(base + j * _CH, _CH)], idxu.at[j])
        pltpu.sync_copy(item_h.at[pl.ds(base + j * _CH, _CH)], idxi.at[j])
    pltpu.sync_copy(label_h.at[pl.ds(base, _BPW)], labv)

    for j in range(_NCH):
        cps = []
        for d in range(_D):
            cps.append(pltpu.async_copy(
                utab_h.at[d].at[idxu.at[j]],
                ubuf.at[d, pl.ds(j * _CH, _CH)], sem))
        for c in cps:
            c.wait()
        cps = []
        for d in range(_D):
            cps.append(pltpu.async_copy(
                itab_h.at[d].at[idxi.at[j]],
                vbuf.at[d, pl.ds(j * _CH, _CH)], sem))
        for c in cps:
            c.wait()

    def g_body(g, sq_acc):
        o = pl.multiple_of(g * _L, _L)
        acc = jnp.zeros((_L,), jnp.float32)
        for d in range(_D):
            acc = acc + ubuf[d, pl.ds(o, _L)] * vbuf[d, pl.ds(o, _L)]
        pred16 = acc + _BIAS
        predv[pl.ds(o, _L)] = pred16
        diff = pred16 - labv[pl.ds(o, _L)]
        rmsev[pl.ds(o, _L)] = jnp.abs(diff)
        return sq_acc + diff * diff

    sq = lax.fori_loop(0, _GROUPS, g_body, jnp.zeros((_L,), jnp.float32))
    sqv[...] = sq

    pltpu.sync_copy(predv, pred_h.at[pl.ds(base, _BPW)])
    pltpu.sync_copy(rmsev, rmse_h.at[pl.ds(base, _BPW)])
    pltpu.sync_copy(sqv, part_h.at[pl.ds(wid * _L, _L)])


def _obj_body(p_ref, o_ref):
    o_ref[0, 0] = jnp.sum(p_ref[...]) * (1.0 / _B)


def kernel(user, item, label, user_table, item_table):
    f32 = jnp.float32
    sc_fn = pl.kernel(
        _sc_body,
        out_type=(
            jax.ShapeDtypeStruct((_B,), f32),
            jax.ShapeDtypeStruct((_B,), f32),
            jax.ShapeDtypeStruct((_NW * _L,), f32),
        ),
        mesh=plsc.VectorSubcoreMesh(core_axis_name="c", subcore_axis_name="s"),
        compiler_params=pltpu.CompilerParams(
            needs_layout_passes=False, use_tc_tiling_on_sc=False),
        scratch_types=[
            pltpu.VMEM((_NCH, _CH), jnp.int32),
            pltpu.VMEM((_NCH, _CH), jnp.int32),
            pltpu.VMEM((_D, _BPW), f32),
            pltpu.VMEM((_D, _BPW), f32),
            pltpu.VMEM((_BPW,), f32),
            pltpu.VMEM((_BPW,), f32),
            pltpu.VMEM((_BPW,), f32),
            pltpu.VMEM((_L,), f32),
            pltpu.SemaphoreType.DMA,
        ],
    )
    pred, rmse, part = sc_fn(
        user, item, label, user_table.T, item_table.T)

    obj2 = pl.pallas_call(
        _obj_body,
        out_shape=jax.ShapeDtypeStruct((1, 1), f32),
        out_specs=pl.BlockSpec(memory_space=pltpu.SMEM),
    )(part.reshape(_NW, _L))

    return (pred, obj2[0, 0], rmse)


# speedup vs baseline: 5.5676x; 5.5676x over previous
import jax
import jax.numpy as jnp
from jax import lax
from jax.experimental import pallas as pl
from jax.experimental.pallas import tpu as pltpu
from jax.experimental.pallas import tpu_sc as plsc

_NC, _NS, _L = 2, 16, 16
_NW = _NC * _NS
_B = 16384
_BPW = _B // _NW
_D = 32
_SR = 128
_CH = 128
_NCH = _BPW // _CH
_GPC = _CH // _L
_BIAS = 3.5


def _sc_body(user_h, item_h, label_h, utab_h, itab_h,
             pred_h, rmse_h, part_h,
             idxu, idxi, sidxu, sidxi, ubuf, vbuf,
             labv, predv, rmsev, sqv, sem):
    wid = lax.axis_index("s") * _NC + lax.axis_index("c")
    base = wid * _BPW

    for j in range(_NCH):
        pltpu.sync_copy(user_h.at[pl.ds(base + j * _CH, _CH)], idxu.at[j])
        pltpu.sync_copy(item_h.at[pl.ds(base + j * _CH, _CH)], idxi.at[j])
    pltpu.sync_copy(label_h.at[pl.ds(base, _BPW)], labv)

    for j in range(_NCH):
        for k in range(_CH // _L):
            s = pl.ds(k * _L, _L)
            sidxu[j, s] = lax.shift_right_logical(idxu[j, s], 2)
            sidxi[j, s] = lax.shift_right_logical(idxi[j, s], 2)

    lane = lax.iota(jnp.int32, _L)

    def chunk(j, ub, vb, sq_acc):
        cu = pltpu.async_copy(utab_h.at[sidxu.at[j]], ub, sem)
        cv = pltpu.async_copy(itab_h.at[sidxi.at[j]], vb, sem)
        cu.wait()
        cv.wait()

        def g_body(g, sq):
            o = pl.multiple_of(j * _CH + g * _L, _L)
            oc = pl.multiple_of(g * _L, _L)
            rows = g * _L + lane
            iu16 = idxu[j, pl.ds(oc, _L)]
            iv16 = idxi[j, pl.ds(oc, _L)]
            cu0 = lax.shift_left(jnp.bitwise_and(iu16, 3), 5)
            cv0 = lax.shift_left(jnp.bitwise_and(iv16, 3), 5)
            acc = jnp.zeros((_L,), jnp.float32)
            for d in range(_D):
                u16 = plsc.load_gather(ub, [rows, cu0 + d])
                v16 = plsc.load_gather(vb, [rows, cv0 + d])
                acc = acc + u16 * v16
            pred16 = acc + _BIAS
            predv[pl.ds(o, _L)] = pred16
            diff = pred16 - labv[pl.ds(o, _L)]
            rmsev[pl.ds(o, _L)] = jnp.abs(diff)
            return sq + diff * diff

        return lax.fori_loop(0, _GPC, g_body, sq_acc)

    sq = jnp.zeros((_L,), jnp.float32)
    for j in range(_NCH):
        sq = chunk(j, ubuf, vbuf, sq)
    sqv[...] = sq

    pltpu.sync_copy(predv, pred_h.at[pl.ds(base, _BPW)])
    pltpu.sync_copy(rmsev, rmse_h.at[pl.ds(base, _BPW)])
    pltpu.sync_copy(sqv, part_h.at[pl.ds(wid * _L, _L)])


def _obj_body(p_ref, o_ref):
    o_ref[0, 0] = jnp.sum(p_ref[...]) * (1.0 / _B)


def kernel(user, item, label, user_table, item_table):
    f32 = jnp.float32
    sc_fn = pl.kernel(
        _sc_body,
        out_type=(
            jax.ShapeDtypeStruct((_B,), f32),
            jax.ShapeDtypeStruct((_B,), f32),
            jax.ShapeDtypeStruct((_NW * _L,), f32),
        ),
        mesh=plsc.VectorSubcoreMesh(core_axis_name="c", subcore_axis_name="s"),
        compiler_params=pltpu.CompilerParams(needs_layout_passes=False),
        scratch_types=[
            pltpu.VMEM((_NCH, _CH), jnp.int32),
            pltpu.VMEM((_NCH, _CH), jnp.int32),
            pltpu.VMEM((_NCH, _CH), jnp.int32),
            pltpu.VMEM((_NCH, _CH), jnp.int32),
            pltpu.VMEM((_CH, _SR), f32),
            pltpu.VMEM((_CH, _SR), f32),
            pltpu.VMEM((_BPW,), f32),
            pltpu.VMEM((_BPW,), f32),
            pltpu.VMEM((_BPW,), f32),
            pltpu.VMEM((_L,), f32),
            pltpu.SemaphoreType.DMA,
        ],
    )
    pred, rmse, part = sc_fn(
        user, item, label,
        user_table.reshape(250000, _SR),
        item_table.reshape(250000, _SR),
    )

    obj2 = pl.pallas_call(
        _obj_body,
        out_shape=jax.ShapeDtypeStruct((1, 1), f32),
        out_specs=pl.BlockSpec(memory_space=pltpu.SMEM),
    )(part.reshape(_NW, _L))

    return (pred, obj2[0, 0], rmse)


# speedup vs baseline: 5.6375x; 1.0126x over previous
import jax
import jax.numpy as jnp
from jax import lax
from jax.experimental import pallas as pl
from jax.experimental.pallas import tpu as pltpu
from jax.experimental.pallas import tpu_sc as plsc

_NC, _NS, _L = 2, 16, 16
_NW = _NC * _NS
_B = 16384
_BPW = _B // _NW
_D = 32
_CH = 128
_NCH = _BPW // _CH
_GROUPS = _BPW // _L
_BIAS = 3.5
_H = _D // _L


def _sc_body(user_h, item_h, label_h, utab_h, itab_h,
             pred_h, rmse_h, part_h,
             idxu, idxi, urows, irows, labv, predv, rmsev, sqv, sem):
    wid = lax.axis_index("s") * _NC + lax.axis_index("c")
    base = wid * _BPW

    for j in range(_NCH):
        pltpu.sync_copy(user_h.at[pl.ds(base + j * _CH, _CH)], idxu.at[j])
        pltpu.sync_copy(item_h.at[pl.ds(base + j * _CH, _CH)], idxi.at[j])
    pltpu.sync_copy(label_h.at[pl.ds(base, _BPW)], labv)

    copies = []
    for j in range(_NCH):
        copies.append(pltpu.async_copy(
            utab_h.at[idxu.at[j]], urows.at[pl.ds(j * _CH, _CH)], sem))
        copies.append(pltpu.async_copy(
            itab_h.at[idxi.at[j]], irows.at[pl.ds(j * _CH, _CH)], sem))
    for c in copies:
        c.wait()

    lane = lax.iota(jnp.int32, _L)
    lane0 = lane == 0

    def row_block(r4, _):
        for q in range(4):
            r = r4 * 4 + q
            p = (urows[r, pl.ds(0, _L)] * irows[r, pl.ds(0, _L)]
                 + urows[r, pl.ds(_L, _L)] * irows[r, pl.ds(_L, _L)])
            s = jnp.sum(p)
            plsc.store_scatter(
                predv, [jnp.full((_L,), r, jnp.int32)],
                jnp.full((_L,), s, jnp.float32), mask=lane0)
        return 0

    lax.fori_loop(0, _BPW // 4, row_block, 0)

    def g_body(g, sq_acc):
        o = pl.multiple_of(g * _L, _L)
        pred16 = predv[pl.ds(o, _L)] + _BIAS
        predv[pl.ds(o, _L)] = pred16
        diff = pred16 - labv[pl.ds(o, _L)]
        rmsev[pl.ds(o, _L)] = jnp.abs(diff)
        return sq_acc + diff * diff

    sq = lax.fori_loop(0, _GROUPS, g_body, jnp.zeros((_L,), jnp.float32))
    sqv[...] = sq

    pltpu.sync_copy(predv, pred_h.at[pl.ds(base, _BPW)])
    pltpu.sync_copy(rmsev, rmse_h.at[pl.ds(base, _BPW)])
    pltpu.sync_copy(sqv, part_h.at[pl.ds(wid * _L, _L)])


def _obj_body(p_ref, o_ref):
    o_ref[0, 0] = jnp.sum(p_ref[...]) * (1.0 / _B)


def kernel(user, item, label, user_table, item_table):
    f32 = jnp.float32
    sc_fn = pl.kernel(
        _sc_body,
        out_type=(
            jax.ShapeDtypeStruct((_B,), f32),
            jax.ShapeDtypeStruct((_B,), f32),
            jax.ShapeDtypeStruct((_NW * _L,), f32),
        ),
        mesh=plsc.VectorSubcoreMesh(core_axis_name="c", subcore_axis_name="s"),
        compiler_params=pltpu.CompilerParams(
            needs_layout_passes=False, use_tc_tiling_on_sc=False),
        scratch_types=[
            pltpu.VMEM((_NCH, _CH), jnp.int32),
            pltpu.VMEM((_NCH, _CH), jnp.int32),
            pltpu.VMEM((_BPW, _D), f32),
            pltpu.VMEM((_BPW, _D), f32),
            pltpu.VMEM((_BPW,), f32),
            pltpu.VMEM((_BPW,), f32),
            pltpu.VMEM((_BPW,), f32),
            pltpu.VMEM((_L,), f32),
            pltpu.SemaphoreType.DMA,
        ],
    )
    pred, rmse, part = sc_fn(user, item, label, user_table, item_table)

    obj2 = pl.pallas_call(
        _obj_body,
        out_shape=jax.ShapeDtypeStruct((1, 1), f32),
        out_specs=pl.BlockSpec(memory_space=pltpu.SMEM),
    )(part.reshape(_NW, _L))

    return (pred, obj2[0, 0], rmse)
